# baseline (speedup 1.0000x reference)
import functools

import jax
import jax.numpy as jnp
from jax import lax
from jax.experimental import pallas as pl
from jax.experimental.pallas import tpu as pltpu
from jax.experimental.pallas import tpu_sc as plsc

BATCH = 16384
K = 64
CH = 128


def _gmf_kernel(uid_hbm, iid_hbm, p2_hbm, q2_hbm, out_hbm,
                uidx_v, iidx_v, utid_v, itid_v, pbuf, qbuf, obuf,
                sem_p0, sem_p1, sem_q0, sem_q1, sem_o0, sem_o1):
    info = plsc.get_sparse_core_info()
    nc = info.num_cores
    nw = nc * info.num_subcores
    lanes = info.num_lanes
    b_per_w = BATCH // nw
    n_chunks = b_per_w // CH

    wid = lax.axis_index("s") * nc + lax.axis_index("c")
    base = wid * b_per_w

    pltpu.sync_copy(uid_hbm.at[pl.ds(base, b_per_w)], uidx_v)
    pltpu.sync_copy(iid_hbm.at[pl.ds(base, b_per_w)], iidx_v)

    for i in range(b_per_w // lanes):
        sl = pl.ds(i * lanes, lanes)
        utid_v[sl] = jax.lax.shift_right_logical(uidx_v[sl], 1)
        itid_v[sl] = jax.lax.shift_right_logical(iidx_v[sl], 1)

    sem_ps = (sem_p0, sem_p1)
    sem_qs = (sem_q0, sem_q1)
    sem_os = (sem_o0, sem_o1)

    def gathers(ch, b):
        sl = pl.ds(ch * CH, CH)
        cp = pltpu.async_copy(p2_hbm.at[utid_v.at[sl]], pbuf.at[b], sem_ps[b])
        cq = pltpu.async_copy(q2_hbm.at[itid_v.at[sl]], qbuf.at[b], sem_qs[b])
        return cp, cq

    gathers(0, 0)
    gathers(1, 1)

    for ch in range(n_chunks):
        b = ch % 2
        sl = pl.ds(ch * CH, CH)
        pltpu.make_async_copy(p2_hbm.at[utid_v.at[sl]], pbuf.at[b],
                              sem_ps[b]).wait()
        pltpu.make_async_copy(q2_hbm.at[itid_v.at[sl]], qbuf.at[b],
                              sem_qs[b]).wait()
        if ch >= 2:
            pltpu.make_async_copy(
                obuf.at[b],
                out_hbm.at[pl.ds(base + (ch - 2) * CH, CH)],
                sem_os[b]).wait()

        def cbody(g, carry):
            off = ch * CH + g * lanes
            uvec = jax.lax.bitwise_and(uidx_v[pl.ds(off, lanes)], 1) * K
            ivec = jax.lax.bitwise_and(iidx_v[pl.ds(off, lanes)], 1) * K
            for l in range(lanes):
                uo = lax.squeeze(lax.slice(uvec, (l,), (l + 1,)), (0,))
                io = lax.squeeze(lax.slice(ivec, (l,), (l + 1,)), (0,))
                r = g * lanes + l
                for k in range(K // lanes):
                    pv = pbuf[b, r, pl.ds(uo + k * lanes, lanes)]
                    qv = qbuf[b, r, pl.ds(io + k * lanes, lanes)]
                    obuf[b, r, pl.ds(k * lanes, lanes)] = pv * qv
            return carry
        lax.fori_loop(0, CH // lanes, cbody, 0)

        pltpu.async_copy(obuf.at[b],
                         out_hbm.at[pl.ds(base + ch * CH, CH)],
                         sem_os[b])
        if ch + 2 < n_chunks:
            gathers(ch + 2, b)

    for b in range(2):
        ch = n_chunks - 2 + b
        pltpu.make_async_copy(obuf.at[b],
                              out_hbm.at[pl.ds(base + ch * CH, CH)],
                              sem_os[b]).wait()


def kernel(user_ids, item_ids, P, Q):
    info = plsc.get_sparse_core_info()
    nw = info.num_cores * info.num_subcores
    b_per_w = BATCH // nw

    p2 = P.reshape(P.shape[0] // 2, 2 * K)
    q2 = Q.reshape(Q.shape[0] // 2, 2 * K)

    mesh = plsc.VectorSubcoreMesh(core_axis_name="c", subcore_axis_name="s")
    run = functools.partial(
        pl.kernel,
        mesh=mesh,
        out_type=jax.ShapeDtypeStruct((BATCH, K), jnp.float32),
        scratch_types=[
            pltpu.VMEM((b_per_w,), jnp.int32),
            pltpu.VMEM((b_per_w,), jnp.int32),
            pltpu.VMEM((b_per_w,), jnp.int32),
            pltpu.VMEM((b_per_w,), jnp.int32),
            pltpu.VMEM((2, CH, 2 * K), jnp.float32),
            pltpu.VMEM((2, CH, 2 * K), jnp.float32),
            pltpu.VMEM((2, CH, K), jnp.float32),
            pltpu.SemaphoreType.DMA,
            pltpu.SemaphoreType.DMA,
            pltpu.SemaphoreType.DMA,
            pltpu.SemaphoreType.DMA,
            pltpu.SemaphoreType.DMA,
            pltpu.SemaphoreType.DMA,
        ],
    )(_gmf_kernel)
    return run(user_ids.astype(jnp.int32), item_ids.astype(jnp.int32), p2, q2)
